# Initial kernel scaffold; baseline (speedup 1.0000x reference)
import jax
import jax.numpy as jnp
from jax.experimental import pallas as pl


def kernel(x, edge_index, edge_attr, conv1_Wr, conv1_Wi, conv1_br, conv1_bi, cheb_W, cheb_b, fc1_Wr, fc1_Wi, fc1_br, fc1_bi, critic_W, critic_b, actor_W, actor_b):
    raise NotImplementedError("write your pallas kernel here")


# speedup vs baseline: 1.5367x; 1.5367x over previous
import functools

import jax
import jax.numpy as jnp
from jax import lax
from jax.experimental import pallas as pl
from jax.experimental.pallas import tpu as pltpu

N = 10000
NP = 10016
F = 16
T = 10
E = 160000
HID = 512
NB = 2504
NG = NP // NB
KB = 2000
NK = 100000 // KB

_blk = lambda: pl.BlockSpec((NB, F), lambda k: (k, 0))
_cst = lambda s: pl.BlockSpec(s, lambda k: (0, 0))


def _front_body(x_ref, wrt_ref, wit_ref, br_ref, bi_ref, d0_ref, d1_ref,
                yr_ref, yi_ref, ur_ref, ui_ref, dinv_ref):
    X = x_ref[...]
    deg = d0_ref[:, 0:1] + d1_ref[:, 0:1]
    dinv = jnp.where(deg > 0, lax.rsqrt(jnp.maximum(deg, 1e-12)), 0.0)
    Yr = jax.nn.relu(jnp.dot(X, wrt_ref[...],
                             preferred_element_type=jnp.float32) + br_ref[...])
    Yi = jax.nn.relu(jnp.dot(X, wit_ref[...],
                             preferred_element_type=jnp.float32) + bi_ref[...])
    yr_ref[...] = Yr
    yi_ref[...] = Yi
    ur_ref[...] = dinv * Yr
    ui_ref[...] = dinv * Yi
    dinv_ref[...] = jnp.broadcast_to(dinv, (NB, F))


def _front(Xp, WrT, WiT, br, bi, D0, D1):
    out = jax.ShapeDtypeStruct((NP, F), jnp.float32)
    return pl.pallas_call(
        _front_body,
        grid=(NG,),
        in_specs=[_blk(), _cst((F, F)), _cst((F, F)), _cst((1, F)),
                  _cst((1, F)), _blk(), _blk()],
        out_specs=(_blk(),) * 5,
        out_shape=(out,) * 5,
    )(Xp, WrT, WiT, br, bi, D0, D1)


def _mid1_body(yr_ref, yi_ref, dinv_ref, a1r_ref, a1i_ref,
               t1r_ref, t1i_ref, u2r_ref, u2i_ref):
    dinv = dinv_ref[...]
    t1r = yr_ref[...] - dinv * a1r_ref[...]
    t1i = yi_ref[...] - dinv * a1i_ref[...]
    t1r_ref[...] = t1r
    t1i_ref[...] = t1i
    u2r_ref[...] = dinv * t1r
    u2i_ref[...] = dinv * t1i


def _mid1(Yr, Yi, Dinv, A1r, A1i):
    out = jax.ShapeDtypeStruct((NP, F), jnp.float32)
    return pl.pallas_call(
        _mid1_body,
        grid=(NG,),
        in_specs=[_blk()] * 5,
        out_specs=(_blk(),) * 4,
        out_shape=(out,) * 4,
    )(Yr, Yi, Dinv, A1r, A1i)


def _mid2_body(yr_ref, yi_ref, t1r_ref, t1i_ref, dinv_ref, a2r_ref, a2i_ref,
               w0_ref, w1_ref, w2_ref, cb_ref, zr_ref, zi_ref):
    dinv = dinv_ref[...]
    t0r = yr_ref[...]
    t0i = yi_ref[...]
    t1r = t1r_ref[...]
    t1i = t1i_ref[...]
    t2r = 2.0 * (t1r - dinv * a2r_ref[...]) - t0r
    t2i = 2.0 * (t1i - dinv * a2i_ref[...]) - t0i
    W0 = w0_ref[...]
    W1 = w1_ref[...]
    W2 = w2_ref[...]
    dot = lambda a, b: jnp.dot(a, b, preferred_element_type=jnp.float32)
    zr_ref[...] = jax.nn.relu(dot(t0r, W0) + dot(t1r, W1) + dot(t2r, W2)
                              + cb_ref[...])
    zi_ref[...] = jax.nn.relu(dot(t0i, W0) + dot(t1i, W1) + dot(t2i, W2))


def _mid2(Yr, Yi, T1r, T1i, Dinv, A2r, A2i, W0p, W1p, W2p, cbp):
    out = jax.ShapeDtypeStruct((NP, F), jnp.float32)
    return pl.pallas_call(
        _mid2_body,
        grid=(NG,),
        in_specs=[_blk()] * 7 + [_cst((F, F))] * 3 + [_cst((1, F))],
        out_specs=(_blk(), _blk()),
        out_shape=(out, out),
    )(Yr, Yi, T1r, T1i, Dinv, A2r, A2i, W0p, W1p, W2p, cbp)


def _fc1_body(z_ref, wr_ref, wi_ref, br_ref, bi_ref,
              cw_ref, cb_ref, aw_ref, ab_ref,
              logits_ref, value_ref, accr_ref, acci_ref):
    k = pl.program_id(0)

    @pl.when(k == 0)
    def _():
        accr_ref[...] = jnp.zeros_like(accr_ref)
        acci_ref[...] = jnp.zeros_like(acci_ref)

    zb = z_ref[...]
    dn = (((0,), (0,)), ((), ()))
    pr = lax.dot_general(zb, wr_ref[...], dn,
                         preferred_element_type=jnp.float32)
    pi = lax.dot_general(zb, wi_ref[...], dn,
                         preferred_element_type=jnp.float32)
    accr_ref[0:2, :] += pr
    acci_ref[0:2, :] += pi

    @pl.when(k == NK - 1)
    def _():
        Pr = accr_ref[0:2, :]
        Pi = acci_ref[0:2, :]
        hr = jax.nn.relu(Pr[0:1, :] - Pi[1:2, :] + br_ref[...])
        hi = jax.nn.relu(Pi[0:1, :] + Pr[1:2, :] + bi_ref[...])
        xc = jnp.concatenate([hr, hi], axis=1)
        value_ref[...] = jnp.dot(xc, cw_ref[...],
                                 preferred_element_type=jnp.float32) + cb_ref[...]
        logits_ref[...] = jnp.dot(xc, aw_ref[...],
                                  preferred_element_type=jnp.float32) + ab_ref[...]


def _fc1(Z2, fWr, fWi, br, bi, cW, cb, aW, ab):
    return pl.pallas_call(
        _fc1_body,
        grid=(NK,),
        in_specs=[
            pl.BlockSpec((KB, 2), lambda k: (k, 0)),
            pl.BlockSpec((KB, HID), lambda k: (k, 0)),
            pl.BlockSpec((KB, HID), lambda k: (k, 0)),
            _cst((1, HID)), _cst((1, HID)),
            _cst((2 * HID, 1)), _cst((1, 1)),
            _cst((2 * HID, 18)), _cst((1, 18)),
        ],
        out_specs=(_cst((1, 18)), _cst((1, 1))),
        out_shape=(
            jax.ShapeDtypeStruct((1, 18), jnp.float32),
            jax.ShapeDtypeStruct((1, 1), jnp.float32),
        ),
        scratch_shapes=[
            pltpu.VMEM((8, HID), jnp.float32),
            pltpu.VMEM((8, HID), jnp.float32),
        ],
    )(Z2, fWr, fWi, br, bi, cW, cb, aW, ab)


def _deg_pass(src, dst, w):
    half = E // 2
    d0 = jax.ops.segment_sum(w[:half], dst[:half], num_segments=NP)
    d1 = jax.ops.segment_sum(w[half:], dst[half:], num_segments=NP)
    z = jnp.zeros((NP, F - 1), jnp.float32)
    return (jnp.concatenate([d0[:, None], z], axis=1),
            jnp.concatenate([d1[:, None], z], axis=1))


def _aw_pass(Ur, Ui, src, dst, w):
    vr = w[:, None] * Ur[src, :]
    vi = w[:, None] * Ui[src, :]
    ar = jax.ops.segment_sum(vr, dst, num_segments=NP)
    ai = jax.ops.segment_sum(vi, dst, num_segments=NP)
    return ar, ai


def kernel(x, edge_index, edge_attr, conv1_Wr, conv1_Wi, conv1_br, conv1_bi,
           cheb_W, cheb_b, fc1_Wr, fc1_Wi, fc1_br, fc1_bi,
           critic_W, critic_b, actor_W, actor_b):
    src = edge_index[0]
    dst = edge_index[1]
    w = edge_attr

    Xp = jnp.pad(x[0], ((0, NP - N), (0, F - T)))
    WrT = jnp.pad(conv1_Wr, ((0, F - T), (0, F - T))).T
    WiT = jnp.pad(conv1_Wi, ((0, F - T), (0, F - T))).T
    brp = jnp.pad(conv1_br, (0, F - T))[None, :]
    bip = jnp.pad(conv1_bi, (0, F - T))[None, :]
    W0p = jnp.pad(cheb_W[0], ((0, F - T), (0, F - T)))
    W1p = jnp.pad(cheb_W[1], ((0, F - T), (0, F - T)))
    W2p = jnp.pad(cheb_W[2], ((0, F - T), (0, F - T)))
    cbp = jnp.pad(cheb_b, (0, F - T))[None, :]

    D0, D1 = _deg_pass(src, dst, w)

    Yr, Yi, U1r, U1i, Dinv = _front(Xp, WrT, WiT, brp, bip, D0, D1)

    A1r, A1i = _aw_pass(U1r, U1i, src, dst, w)
    T1r, T1i, U2r, U2i = _mid1(Yr, Yi, Dinv, A1r, A1i)

    A2r, A2i = _aw_pass(U2r, U2i, src, dst, w)
    Zr, Zi = _mid2(Yr, Yi, T1r, T1i, Dinv, A2r, A2i, W0p, W1p, W2p, cbp)

    zr_flat = Zr[:N, :T].reshape(-1)
    zi_flat = Zi[:N, :T].reshape(-1)
    Z2 = jnp.stack([zr_flat, zi_flat], axis=1)
    logits, value = _fc1(Z2, fc1_Wr, fc1_Wi,
                         fc1_br[None, :], fc1_bi[None, :],
                         critic_W, critic_b[None, :], actor_W, actor_b[None, :])
    return (logits, value)


# speedup vs baseline: 8.9133x; 5.8004x over previous
import functools

import jax
import jax.numpy as jnp
from jax import lax
from jax.experimental import pallas as pl
from jax.experimental.pallas import tpu as pltpu

N = 10000
NP = 10112
F = 16
T = 10
E = 160000
HID = 512
NB = 2528
NG = NP // NB
KB = 2000
NK = 100000 // KB

_blk = lambda: pl.BlockSpec((NB, F), lambda k: (k, 0))
_cst = lambda s: pl.BlockSpec(s, lambda k: (0, 0))


def _front_body(x_ref, wrt_ref, wit_ref, br_ref, bi_ref, d0_ref,
                yr_ref, yi_ref, ur_ref, ui_ref, dinv_ref):
    X = x_ref[...]
    deg = d0_ref[:, 0:1]
    dinv = jnp.where(deg > 0, lax.rsqrt(jnp.maximum(deg, 1e-12)), 0.0)
    Yr = jax.nn.relu(jnp.dot(X, wrt_ref[...],
                             preferred_element_type=jnp.float32) + br_ref[...])
    Yi = jax.nn.relu(jnp.dot(X, wit_ref[...],
                             preferred_element_type=jnp.float32) + bi_ref[...])
    yr_ref[...] = Yr
    yi_ref[...] = Yi
    ur_ref[...] = dinv * Yr
    ui_ref[...] = dinv * Yi
    dinv_ref[...] = jnp.broadcast_to(dinv, (NB, F))


def _front(Xp, WrT, WiT, br, bi, D0):
    out = jax.ShapeDtypeStruct((NP, F), jnp.float32)
    return pl.pallas_call(
        _front_body,
        grid=(NG,),
        in_specs=[_blk(), _cst((F, F)), _cst((F, F)), _cst((1, F)),
                  _cst((1, F)), _blk()],
        out_specs=(_blk(),) * 5,
        out_shape=(out,) * 5,
    )(Xp, WrT, WiT, br, bi, D0)


def _mid1_body(yr_ref, yi_ref, dinv_ref, a1r_ref, a1i_ref,
               t1r_ref, t1i_ref, u2r_ref, u2i_ref):
    dinv = dinv_ref[...]
    t1r = yr_ref[...] - dinv * a1r_ref[...]
    t1i = yi_ref[...] - dinv * a1i_ref[...]
    t1r_ref[...] = t1r
    t1i_ref[...] = t1i
    u2r_ref[...] = dinv * t1r
    u2i_ref[...] = dinv * t1i


def _mid1(Yr, Yi, Dinv, A1r, A1i):
    out = jax.ShapeDtypeStruct((NP, F), jnp.float32)
    return pl.pallas_call(
        _mid1_body,
        grid=(NG,),
        in_specs=[_blk()] * 5,
        out_specs=(_blk(),) * 4,
        out_shape=(out,) * 4,
    )(Yr, Yi, Dinv, A1r, A1i)


def _mid2_body(yr_ref, yi_ref, t1r_ref, t1i_ref, dinv_ref, a2r_ref, a2i_ref,
               w0_ref, w1_ref, w2_ref, cb_ref, zr_ref, zi_ref):
    dinv = dinv_ref[...]
    t0r = yr_ref[...]
    t0i = yi_ref[...]
    t1r = t1r_ref[...]
    t1i = t1i_ref[...]
    t2r = 2.0 * (t1r - dinv * a2r_ref[...]) - t0r
    t2i = 2.0 * (t1i - dinv * a2i_ref[...]) - t0i
    W0 = w0_ref[...]
    W1 = w1_ref[...]
    W2 = w2_ref[...]
    dot = lambda a, b: jnp.dot(a, b, preferred_element_type=jnp.float32)
    zr_ref[...] = jax.nn.relu(dot(t0r, W0) + dot(t1r, W1) + dot(t2r, W2)
                              + cb_ref[...])
    zi_ref[...] = jax.nn.relu(dot(t0i, W0) + dot(t1i, W1) + dot(t2i, W2))


def _mid2(Yr, Yi, T1r, T1i, Dinv, A2r, A2i, W0p, W1p, W2p, cbp):
    out = jax.ShapeDtypeStruct((NP, F), jnp.float32)
    return pl.pallas_call(
        _mid2_body,
        grid=(NG,),
        in_specs=[_blk()] * 7 + [_cst((F, F))] * 3 + [_cst((1, F))],
        out_specs=(_blk(), _blk()),
        out_shape=(out, out),
    )(Yr, Yi, T1r, T1i, Dinv, A2r, A2i, W0p, W1p, W2p, cbp)


def _fc1_body(z_ref, wr_ref, wi_ref, br_ref, bi_ref,
              cw_ref, cb_ref, aw_ref, ab_ref,
              logits_ref, value_ref, accr_ref, acci_ref):
    k = pl.program_id(0)

    @pl.when(k == 0)
    def _():
        accr_ref[...] = jnp.zeros_like(accr_ref)
        acci_ref[...] = jnp.zeros_like(acci_ref)

    zb = z_ref[...]
    dn = (((0,), (0,)), ((), ()))
    pr = lax.dot_general(zb, wr_ref[...], dn,
                         preferred_element_type=jnp.float32)
    pi = lax.dot_general(zb, wi_ref[...], dn,
                         preferred_element_type=jnp.float32)
    accr_ref[0:2, :] += pr
    acci_ref[0:2, :] += pi

    @pl.when(k == NK - 1)
    def _():
        Pr = accr_ref[0:2, :]
        Pi = acci_ref[0:2, :]
        hr = jax.nn.relu(Pr[0:1, :] - Pi[1:2, :] + br_ref[...])
        hi = jax.nn.relu(Pi[0:1, :] + Pr[1:2, :] + bi_ref[...])
        xc = jnp.concatenate([hr, hi], axis=1)
        value_ref[...] = jnp.dot(xc, cw_ref[...],
                                 preferred_element_type=jnp.float32) + cb_ref[...]
        logits_ref[...] = jnp.dot(xc, aw_ref[...],
                                  preferred_element_type=jnp.float32) + ab_ref[...]


def _fc1(Z2, fWr, fWi, br, bi, cW, cb, aW, ab):
    return pl.pallas_call(
        _fc1_body,
        grid=(NK,),
        in_specs=[
            pl.BlockSpec((KB, 2), lambda k: (k, 0)),
            pl.BlockSpec((KB, HID), lambda k: (k, 0)),
            pl.BlockSpec((KB, HID), lambda k: (k, 0)),
            _cst((1, HID)), _cst((1, HID)),
            _cst((2 * HID, 1)), _cst((1, 1)),
            _cst((2 * HID, 18)), _cst((1, 18)),
        ],
        out_specs=(_cst((1, 18)), _cst((1, 1))),
        out_shape=(
            jax.ShapeDtypeStruct((1, 18), jnp.float32),
            jax.ShapeDtypeStruct((1, 1), jnp.float32),
        ),
        scratch_shapes=[
            pltpu.VMEM((8, HID), jnp.float32),
            pltpu.VMEM((8, HID), jnp.float32),
        ],
    )(Z2, fWr, fWi, br, bi, cW, cb, aW, ab)


from jax.experimental.pallas import tpu_sc as plsc

_MESH = plsc.VectorSubcoreMesh(core_axis_name="c", subcore_axis_name="s")
CCH = 400
NCHA = 10000 // CCH
GP = CCH // 16
NPT = NP // 16
ES = 160256
EPT = ES // 32
DTAIL = EPT - 12 * CCH


def _aw_body(uu_hbm, src_hbm, dst_hbm, w_hbm, z_hbm,
             or_hbm, oi_hbm, sidx, didx, wv, G, R, ACC, sem):
    c = lax.axis_index("c")
    s = lax.axis_index("s")
    pltpu.sync_copy(z_hbm, ACC.at[pl.ds(s * NPT, NPT)])
    plsc.subcore_barrier()
    uoff = c * NP

    def chunk(k, _):
        base = s * 10000 + k * CCH
        pltpu.sync_copy(src_hbm.at[pl.ds(base, CCH)], sidx)
        pltpu.sync_copy(dst_hbm.at[pl.ds(base, CCH)], didx)
        pltpu.sync_copy(w_hbm.at[pl.ds(base, CCH)], wv)
        for g in range(GP):
            sl16 = pl.ds(g * 16, 16)
            sidx[sl16] = sidx[sl16] + uoff
        pltpu.async_copy(uu_hbm.at[sidx], G, sem).wait()

        for g in range(GP):
            wv16 = wv[pl.ds(g * 16, 16)]
            for j in range(16):
                e = g * 16 + j
                R[e, :] = G[e, :] * wv16[j]
        pltpu.sync_copy(R, ACC.at[didx], add=True)
        return 0

    lax.fori_loop(0, NCHA, chunk, 0, unroll=False)
    plsc.subcore_barrier()
    sl = pl.ds(s * NPT, NPT)

    @pl.when(c == 0)
    def _():
        pltpu.sync_copy(ACC.at[sl], or_hbm.at[sl])

    @pl.when(c == 1)
    def _():
        pltpu.sync_copy(ACC.at[sl], oi_hbm.at[sl])


def _aw_pass(Ur, Ui, src, dst, w, zrows):
    f = pl.kernel(
        _aw_body,
        out_type=(jax.ShapeDtypeStruct((NP, F), jnp.float32),) * 2,
        mesh=_MESH,
        scratch_types=[
            pltpu.VMEM((CCH,), jnp.int32),
            pltpu.VMEM((CCH,), jnp.int32),
            pltpu.VMEM((CCH,), jnp.float32),
            pltpu.VMEM((CCH, F), jnp.float32),
            pltpu.VMEM((CCH, F), jnp.float32),
            pltpu.VMEM_SHARED((NP, F), jnp.float32),
            pltpu.SemaphoreType.DMA,
        ],
        compiler_params=pltpu.CompilerParams(use_tc_tiling_on_sc=False),
    )
    UU = jnp.concatenate([Ur, Ui], axis=0)
    return f(UU, src, dst, w, zrows)


def kernel(x, edge_index, edge_attr, conv1_Wr, conv1_Wi, conv1_br, conv1_bi,
           cheb_W, cheb_b, fc1_Wr, fc1_Wi, fc1_br, fc1_bi,
           critic_W, critic_b, actor_W, actor_b):
    src = edge_index[0]
    dst = edge_index[1]
    w = edge_attr

    Xp = jnp.pad(x[0], ((0, NP - N), (0, F - T)))
    WrT = jnp.pad(conv1_Wr, ((0, F - T), (0, F - T))).T
    WiT = jnp.pad(conv1_Wi, ((0, F - T), (0, F - T))).T
    brp = jnp.pad(conv1_br, (0, F - T))[None, :]
    bip = jnp.pad(conv1_bi, (0, F - T))[None, :]
    W0p = jnp.pad(cheb_W[0], ((0, F - T), (0, F - T)))
    W1p = jnp.pad(cheb_W[1], ((0, F - T), (0, F - T)))
    W2p = jnp.pad(cheb_W[2], ((0, F - T), (0, F - T)))
    cbp = jnp.pad(cheb_b, (0, F - T))[None, :]

    zrows = jnp.zeros((NPT, F), jnp.float32)
    ones1 = jnp.zeros((NP, F), jnp.float32).at[:, 0].set(1.0)

    D0, _ = _aw_pass(ones1, ones1, src, dst, w, zrows)

    Yr, Yi, U1r, U1i, Dinv = _front(Xp, WrT, WiT, brp, bip, D0)

    A1r, A1i = _aw_pass(U1r, U1i, src, dst, w, zrows)
    T1r, T1i, U2r, U2i = _mid1(Yr, Yi, Dinv, A1r, A1i)

    A2r, A2i = _aw_pass(U2r, U2i, src, dst, w, zrows)
    Zr, Zi = _mid2(Yr, Yi, T1r, T1i, Dinv, A2r, A2i, W0p, W1p, W2p, cbp)

    zr_flat = Zr[:N, :T].reshape(-1)
    zi_flat = Zi[:N, :T].reshape(-1)
    Z2 = jnp.stack([zr_flat, zi_flat], axis=1)
    logits, value = _fc1(Z2, fc1_Wr, fc1_Wi,
                         fc1_br[None, :], fc1_bi[None, :],
                         critic_W, critic_b[None, :], actor_W, actor_b[None, :])
    return (logits, value)
